# baseline (speedup 1.0000x reference)
import dataclasses
import functools

import jax
import jax.numpy as jnp
from jax import lax
from jax.experimental import pallas as pl
from jax.experimental.pallas import tpu as pltpu
from jax.experimental.pallas import tpu_sc as plsc

B = 16
N = 512 * 512
NB = 1024
SCALE = 64.0
VMAX = (NB - 1) / SCALE
NLANE = 16
NWORK = 32
HALF = N // 2
WIN = 8192
NWIN = HALF // WIN
CHUNK = 2048
NCH = HALF // CHUNK
NSLOT = NB + NCH
STRIDE = NSLOT + 1
TOTF = float(B * N)


def _sc_pass_kernel(yp_hbm, yt_hbm, cnt_o, fsum_o, npos_o, ypb0, ypb1, ytb0,
                    ytb1, cntv, fsumv, nposv, sems):
    wid = lax.axis_index("s") * 2 + lax.axis_index("c")
    base = wid * HALF

    zeros16 = jnp.zeros((NLANE,), jnp.float32)
    ones16 = jnp.ones((NLANE,), jnp.float32)
    lanebase = lax.iota(jnp.int32, NLANE) * STRIDE

    @pl.loop(0, STRIDE * NLANE, step=NLANE)
    def _zero(i):
        i = pl.multiple_of(i, NLANE)
        cntv[pl.ds(i, NLANE)] = zeros16
        fsumv[pl.ds(i, NLANE)] = zeros16

    def _compute_window(w, buf_idx, np_acc):
        ypb = ypb1 if buf_idx else ypb0
        ytb = ytb1 if buf_idx else ytb0

        def body(j, acc):
            sl = pl.ds(pl.multiple_of(j * NLANE, NLANE), NLANE)
            ypv = ypb[sl]
            ytv = ytb[sl]
            posm = ytv > 0.0
            d = ypv - ytv
            ad = jnp.abs(d)
            sl1 = jnp.where(ad < 1.0, (0.5 * d) * d, ad - 0.5)
            lc = jnp.where(posm, 0.0, ad)
            bidx = (jnp.minimum(lc, VMAX) * SCALE).astype(jnp.int32)
            chunk_slot = NB + (w * WIN + j * NLANE) // CHUNK
            slot = jnp.where(lc == 0.0, chunk_slot, bidx)
            idx16 = lanebase + slot
            plsc.addupdate_scatter(cntv, [idx16], ones16)
            plsc.addupdate_scatter(fsumv, [idx16], sl1)
            return acc + jnp.where(posm, 1.0, 0.0)

        return plsc.parallel_loop(0, WIN // NLANE, unroll=16, carry=np_acc)(body)

    def _start_copies(w, buf_idx):
        off = pl.multiple_of(base + w * WIN, WIN)
        ypb = ypb1 if buf_idx else ypb0
        ytb = ytb1 if buf_idx else ytb0
        pltpu.make_async_copy(yp_hbm.at[pl.ds(off, WIN)], ypb,
                              sems.at[buf_idx, 0]).start()
        pltpu.make_async_copy(yt_hbm.at[pl.ds(off, WIN)], ytb,
                              sems.at[buf_idx, 1]).start()

    def _wait_copies(w, buf_idx):
        off = pl.multiple_of(base + w * WIN, WIN)
        ypb = ypb1 if buf_idx else ypb0
        ytb = ytb1 if buf_idx else ytb0
        pltpu.make_async_copy(yp_hbm.at[pl.ds(off, WIN)], ypb,
                              sems.at[buf_idx, 0]).wait()
        pltpu.make_async_copy(yt_hbm.at[pl.ds(off, WIN)], ytb,
                              sems.at[buf_idx, 1]).wait()

    _start_copies(0, 0)
    _start_copies(1, 1)

    def _outer(i, acc):
        w = i * 2
        _wait_copies(w, 0)
        acc = _compute_window(w, 0, acc)
        _start_copies(w + 2, 0)
        _wait_copies(w + 1, 1)
        acc = _compute_window(w + 1, 1, acc)
        _start_copies(w + 3, 1)
        return acc

    np_acc = lax.fori_loop(0, NWIN // 2 - 1, _outer, zeros16)
    w_last = NWIN - 2
    _wait_copies(w_last, 0)
    np_acc = _compute_window(w_last, 0, np_acc)
    _wait_copies(w_last + 1, 1)
    np_acc = _compute_window(w_last + 1, 1, np_acc)
    nposv[...] = np_acc

    pltpu.sync_copy(cntv, cnt_o.at[wid])
    pltpu.sync_copy(fsumv, fsum_o.at[wid])
    pltpu.sync_copy(nposv, npos_o.at[wid])


def _sc_pass(yp_flat, yt_flat):
    mesh = plsc.VectorSubcoreMesh(core_axis_name="c", subcore_axis_name="s")
    out_type = (
        jax.ShapeDtypeStruct((NWORK, NLANE * STRIDE), jnp.float32),
        jax.ShapeDtypeStruct((NWORK, NLANE * STRIDE), jnp.float32),
        jax.ShapeDtypeStruct((NWORK, NLANE), jnp.float32),
    )
    scratch_types = [
        pltpu.VMEM((WIN,), jnp.float32),
        pltpu.VMEM((WIN,), jnp.float32),
        pltpu.VMEM((WIN,), jnp.float32),
        pltpu.VMEM((WIN,), jnp.float32),
        pltpu.VMEM((NLANE * STRIDE,), jnp.float32),
        pltpu.VMEM((NLANE * STRIDE,), jnp.float32),
        pltpu.VMEM((NLANE,), jnp.float32),
        pltpu.SemaphoreType.DMA((2, 2)),
    ]
    cp = pltpu.CompilerParams()
    if "needs_layout_passes" in pltpu.CompilerParams.__dataclass_fields__:
        cp = dataclasses.replace(cp, needs_layout_passes=False)
    fn = pl.kernel(_sc_pass_kernel, out_type=out_type, mesh=mesh,
                   scratch_types=scratch_types, compiler_params=cp)
    return fn(yp_flat, yt_flat)


def _finish_kernel(cnt_ref, fsum_ref, npos_ref, out_ref):
    f32 = jnp.float32
    xc = cnt_ref[...]
    xf = fsum_ref[...]
    npos_in = npos_ref[...]

    nrows_in = NWORK * NLANE
    r_i = lax.broadcasted_iota(jnp.int32, (B, nrows_in), 0)
    i_i = lax.broadcasted_iota(jnp.int32, (B, nrows_in), 1)
    mrow = (i_i // (2 * NLANE) == r_i).astype(f32)
    t_i = lax.broadcasted_iota(jnp.int32, (NWORK, nrows_in), 0)
    i2_i = lax.broadcasted_iota(jnp.int32, (NWORK, nrows_in), 1)
    mhalf = (i2_i // NLANE == t_i).astype(f32)

    r2_i = lax.broadcasted_iota(jnp.int32, (B, NWORK), 0)
    h_i = lax.broadcasted_iota(jnp.int32, (B, NWORK), 1)
    mpair = (h_i // 2 == r2_i).astype(f32)
    e_t = lax.broadcasted_iota(jnp.int32, (NWORK, B), 0)
    e_r = lax.broadcasted_iota(jnp.int32, (NWORK, B), 1)
    expand = (e_r == e_t // 2).astype(f32)

    hp = jax.lax.Precision.HIGHEST
    full = jnp.dot(mrow, xc, precision=hp)
    fullf = jnp.dot(mrow, xf, precision=hp)
    cnt = full[:, :NB]
    fsum = fullf[:, :NB]
    zh = jnp.dot(mhalf, xc, precision=hp)[:, NB:NB + NCH]
    zsh = jnp.dot(mhalf, xf, precision=hp)[:, NB:NB + NCH]

    npos = jnp.dot(mpair, jnp.sum(npos_in, axis=1, keepdims=True),
                   precision=hp)

    k = jnp.minimum(2.0 * npos, float(N - 1))

    b_a = lax.broadcasted_iota(jnp.int32, (NB, NB), 0)
    b_b = lax.broadcasted_iota(jnp.int32, (NB, NB), 1)
    tsuf = (b_a >= b_b).astype(f32)
    suffix = jnp.dot(cnt, tsuf, precision=hp)

    iota_b = lax.broadcasted_iota(jnp.int32, (B, NB), 1).astype(f32)
    beta = jnp.max(jnp.where(suffix >= k, iota_b, 0.0), axis=1,
                   keepdims=True)

    gt = iota_b > beta
    c_gt = jnp.sum(jnp.where(gt, cnt, 0.0), axis=1, keepdims=True)
    s_gt = jnp.sum(jnp.where(gt, fsum, 0.0), axis=1, keepdims=True)
    at = iota_b == beta
    cnt_b = jnp.sum(jnp.where(at, cnt, 0.0), axis=1, keepdims=True)
    sum_b = jnp.sum(jnp.where(at, fsum, 0.0), axis=1, keepdims=True)
    m = k - c_gt
    s_tie = jnp.where(m > 0.0, m * sum_b / jnp.maximum(cnt_b, 1.0), 0.0)
    s_neg_hi = s_gt + s_tie

    ge1 = iota_b >= 1.0
    c_gt0 = jnp.sum(jnp.where(ge1, cnt, 0.0), axis=1, keepdims=True)
    s_gt0 = jnp.sum(jnp.where(ge1, fsum, 0.0), axis=1, keepdims=True)
    nzero_h = jnp.sum(zh, axis=1, keepdims=True)
    nzero = jnp.dot(mpair, nzero_h, precision=hp)
    zsum = jnp.dot(mpair, jnp.sum(zsh, axis=1, keepdims=True), precision=hp)
    ns = cnt[:, :1]
    fs_small = fsum[:, :1]
    m0 = jnp.maximum(k - c_gt0, 0.0)
    q = jnp.minimum(m0, ns)
    s_small = q * fs_small / jnp.maximum(ns, 1.0)
    mz = m0 - q

    c_a = lax.broadcasted_iota(jnp.int32, (NCH, NCH), 0)
    c_b = lax.broadcasted_iota(jnp.int32, (NCH, NCH), 1)
    tpre = (c_a < c_b).astype(f32)
    prefix = jnp.dot(zh, tpre, precision=hp)
    t2_i = lax.broadcasted_iota(jnp.int32, (NWORK, NWORK), 0)
    tp_i = lax.broadcasted_iota(jnp.int32, (NWORK, NWORK), 1)
    shift_prev = ((t2_i % 2 == 1) & (tp_i == t2_i - 1)).astype(f32)
    prev_total = jnp.dot(shift_prev, nzero_h, precision=hp)
    prefix = prefix + prev_total
    mz_h = jnp.dot(expand, mz, precision=hp)
    selc = jnp.clip(mz_h - prefix, 0.0, zh) / jnp.maximum(zh, 1.0)
    s_z_h = jnp.sum(selc * zsh, axis=1, keepdims=True)
    s_z = jnp.dot(mpair, s_z_h, precision=hp)
    s_neg0 = s_gt0 + s_small + s_z

    s_neg = jnp.where(beta == 0.0, s_neg0, s_neg_hi)

    pos_cnt = jnp.maximum(jnp.sum(npos), 1.0)
    neg_cnt = jnp.maximum(jnp.sum(k), 1.0)
    loss = 2.0 * (jnp.sum(zsum) / pos_cnt) + jnp.sum(s_neg) / neg_cnt
    out_ref[0, 0] = loss


def _finish(cnt, fsum, npos):
    return pl.pallas_call(
        _finish_kernel,
        out_shape=jax.ShapeDtypeStruct((1, 1), jnp.float32),
        out_specs=pl.BlockSpec(memory_space=pltpu.SMEM),
    )(cnt.reshape(NWORK * NLANE, STRIDE), fsum.reshape(NWORK * NLANE, STRIDE),
      npos)


def kernel(y_pred, y_true):
    yp = y_pred.reshape(-1)
    yt = y_true.reshape(-1)
    cnt, fsum, npos = _sc_pass(yp, yt)
    out = _finish(cnt, fsum, npos)
    return jnp.reshape(out, ())


# speedup vs baseline: 1.2588x; 1.2588x over previous
import dataclasses
import functools

import jax
import jax.numpy as jnp
from jax import lax
from jax.experimental import pallas as pl
from jax.experimental.pallas import tpu as pltpu
from jax.experimental.pallas import tpu_sc as plsc

B = 16
N = 512 * 512
NB = 1024
SCALE = 64.0
VMAX = (NB - 1) / SCALE
NLANE = 16
NWORK = 32
HALF = N // 2
WIN = 8192
NWIN = HALF // WIN
CHUNK = 2048
NCH = HALF // CHUNK
NSLOT = NB + NCH
STRIDE = NSLOT + 1
TOTF = float(B * N)


def _sc_pass_kernel(yp_hbm, yt_hbm, cnt_o, fsum_o, npos_o, ypb0, ypb1, ytb0,
                    ytb1, cntv, fsumv, nposv, sems):
    wid = lax.axis_index("s") * 2 + lax.axis_index("c")
    base = wid * HALF

    zeros16 = jnp.zeros((NLANE,), jnp.float32)
    ones16 = jnp.ones((NLANE,), jnp.float32)
    lanebase = lax.iota(jnp.int32, NLANE) * STRIDE

    def _compute_window(w, buf_idx, np_acc):
        ypb = ypb1 if buf_idx else ypb0
        ytb = ytb1 if buf_idx else ytb0

        def body(j, acc):
            sl = pl.ds(pl.multiple_of(j * NLANE, NLANE), NLANE)
            ypv = ypb[sl]
            ytv = ytb[sl]
            posm = ytv > 0.0
            d = ypv - ytv
            ad = jnp.abs(d)
            sl1 = jnp.where(ad < 1.0, (0.5 * d) * d, ad - 0.5)
            lc = jnp.where(posm, 0.0, ad)
            bidx = (jnp.minimum(lc, VMAX) * SCALE).astype(jnp.int32)
            chunk_slot = NB + (w * WIN + j * NLANE) // CHUNK
            slot = jnp.where(lc == 0.0, chunk_slot, bidx)
            idx16 = lanebase + slot
            plsc.addupdate_scatter(cntv, [idx16], ones16)
            plsc.addupdate_scatter(fsumv, [idx16], sl1)
            return acc + jnp.where(posm, 1.0, 0.0)

        return plsc.parallel_loop(0, WIN // NLANE, unroll=8, carry=np_acc)(body)

    def _start_copies(w, buf_idx):
        off = pl.multiple_of(base + w * WIN, WIN)
        ypb = ypb1 if buf_idx else ypb0
        ytb = ytb1 if buf_idx else ytb0
        pltpu.make_async_copy(yp_hbm.at[pl.ds(off, WIN)], ypb,
                              sems.at[buf_idx, 0]).start()
        pltpu.make_async_copy(yt_hbm.at[pl.ds(off, WIN)], ytb,
                              sems.at[buf_idx, 1]).start()

    def _wait_copies(w, buf_idx):
        off = pl.multiple_of(base + w * WIN, WIN)
        ypb = ypb1 if buf_idx else ypb0
        ytb = ytb1 if buf_idx else ytb0
        pltpu.make_async_copy(yp_hbm.at[pl.ds(off, WIN)], ypb,
                              sems.at[buf_idx, 0]).wait()
        pltpu.make_async_copy(yt_hbm.at[pl.ds(off, WIN)], ytb,
                              sems.at[buf_idx, 1]).wait()

    _start_copies(0, 0)
    _start_copies(1, 1)

    @pl.loop(0, STRIDE * NLANE, step=NLANE)
    def _zero(i):
        i = pl.multiple_of(i, NLANE)
        cntv[pl.ds(i, NLANE)] = zeros16
        fsumv[pl.ds(i, NLANE)] = zeros16

    def _outer(i, acc):
        w = i * 2
        _wait_copies(w, 0)
        acc = _compute_window(w, 0, acc)
        _start_copies(w + 2, 0)
        _wait_copies(w + 1, 1)
        acc = _compute_window(w + 1, 1, acc)
        _start_copies(w + 3, 1)
        return acc

    np_acc = lax.fori_loop(0, NWIN // 2 - 1, _outer, zeros16)
    w_last = NWIN - 2
    _wait_copies(w_last, 0)
    np_acc = _compute_window(w_last, 0, np_acc)
    _wait_copies(w_last + 1, 1)
    np_acc = _compute_window(w_last + 1, 1, np_acc)
    nposv[...] = np_acc

    pltpu.sync_copy(cntv, cnt_o.at[wid])
    pltpu.sync_copy(fsumv, fsum_o.at[wid])
    pltpu.sync_copy(nposv, npos_o.at[wid])


def _sc_pass(yp_flat, yt_flat):
    mesh = plsc.VectorSubcoreMesh(core_axis_name="c", subcore_axis_name="s")
    out_type = (
        jax.ShapeDtypeStruct((NWORK, NLANE * STRIDE), jnp.float32),
        jax.ShapeDtypeStruct((NWORK, NLANE * STRIDE), jnp.float32),
        jax.ShapeDtypeStruct((NWORK, NLANE), jnp.float32),
    )
    scratch_types = [
        pltpu.VMEM((WIN,), jnp.float32),
        pltpu.VMEM((WIN,), jnp.float32),
        pltpu.VMEM((WIN,), jnp.float32),
        pltpu.VMEM((WIN,), jnp.float32),
        pltpu.VMEM((NLANE * STRIDE,), jnp.float32),
        pltpu.VMEM((NLANE * STRIDE,), jnp.float32),
        pltpu.VMEM((NLANE,), jnp.float32),
        pltpu.SemaphoreType.DMA((2, 2)),
    ]
    cp = pltpu.CompilerParams()
    if "needs_layout_passes" in pltpu.CompilerParams.__dataclass_fields__:
        cp = dataclasses.replace(cp, needs_layout_passes=False)
    fn = pl.kernel(_sc_pass_kernel, out_type=out_type, mesh=mesh,
                   scratch_types=scratch_types, compiler_params=cp)
    return fn(yp_flat, yt_flat)


def _finish_kernel(cnt_ref, fsum_ref, npos_ref, out_ref):
    f32 = jnp.float32
    xc = cnt_ref[...]
    xf = fsum_ref[...]
    npos_in = npos_ref[...]

    nrows_in = NWORK * NLANE
    r_i = lax.broadcasted_iota(jnp.int32, (B, nrows_in), 0)
    i_i = lax.broadcasted_iota(jnp.int32, (B, nrows_in), 1)
    mrow = (i_i // (2 * NLANE) == r_i).astype(f32)
    t_i = lax.broadcasted_iota(jnp.int32, (NWORK, nrows_in), 0)
    i2_i = lax.broadcasted_iota(jnp.int32, (NWORK, nrows_in), 1)
    mhalf = (i2_i // NLANE == t_i).astype(f32)

    r2_i = lax.broadcasted_iota(jnp.int32, (B, NWORK), 0)
    h_i = lax.broadcasted_iota(jnp.int32, (B, NWORK), 1)
    mpair = (h_i // 2 == r2_i).astype(f32)
    e_t = lax.broadcasted_iota(jnp.int32, (NWORK, B), 0)
    e_r = lax.broadcasted_iota(jnp.int32, (NWORK, B), 1)
    expand = (e_r == e_t // 2).astype(f32)

    hp = jax.lax.Precision.HIGHEST
    full = jnp.dot(mrow, xc, precision=hp)
    fullf = jnp.dot(mrow, xf, precision=hp)
    cnt = full[:, :NB]
    fsum = fullf[:, :NB]
    zh = jnp.dot(mhalf, xc, precision=hp)[:, NB:NB + NCH]
    zsh = jnp.dot(mhalf, xf, precision=hp)[:, NB:NB + NCH]

    npos = jnp.dot(mpair, jnp.sum(npos_in, axis=1, keepdims=True),
                   precision=hp)

    k = jnp.minimum(2.0 * npos, float(N - 1))

    b_a = lax.broadcasted_iota(jnp.int32, (NB, NB), 0)
    b_b = lax.broadcasted_iota(jnp.int32, (NB, NB), 1)
    tsuf = (b_a >= b_b).astype(f32)
    suffix = jnp.dot(cnt, tsuf, precision=hp)

    iota_b = lax.broadcasted_iota(jnp.int32, (B, NB), 1).astype(f32)
    beta = jnp.max(jnp.where(suffix >= k, iota_b, 0.0), axis=1,
                   keepdims=True)

    gt = iota_b > beta
    c_gt = jnp.sum(jnp.where(gt, cnt, 0.0), axis=1, keepdims=True)
    s_gt = jnp.sum(jnp.where(gt, fsum, 0.0), axis=1, keepdims=True)
    at = iota_b == beta
    cnt_b = jnp.sum(jnp.where(at, cnt, 0.0), axis=1, keepdims=True)
    sum_b = jnp.sum(jnp.where(at, fsum, 0.0), axis=1, keepdims=True)
    m = k - c_gt
    s_tie = jnp.where(m > 0.0, m * sum_b / jnp.maximum(cnt_b, 1.0), 0.0)
    s_neg_hi = s_gt + s_tie

    ge1 = iota_b >= 1.0
    c_gt0 = jnp.sum(jnp.where(ge1, cnt, 0.0), axis=1, keepdims=True)
    s_gt0 = jnp.sum(jnp.where(ge1, fsum, 0.0), axis=1, keepdims=True)
    nzero_h = jnp.sum(zh, axis=1, keepdims=True)
    nzero = jnp.dot(mpair, nzero_h, precision=hp)
    zsum = jnp.dot(mpair, jnp.sum(zsh, axis=1, keepdims=True), precision=hp)
    ns = cnt[:, :1]
    fs_small = fsum[:, :1]
    m0 = jnp.maximum(k - c_gt0, 0.0)
    q = jnp.minimum(m0, ns)
    s_small = q * fs_small / jnp.maximum(ns, 1.0)
    mz = m0 - q

    c_a = lax.broadcasted_iota(jnp.int32, (NCH, NCH), 0)
    c_b = lax.broadcasted_iota(jnp.int32, (NCH, NCH), 1)
    tpre = (c_a < c_b).astype(f32)
    prefix = jnp.dot(zh, tpre, precision=hp)
    t2_i = lax.broadcasted_iota(jnp.int32, (NWORK, NWORK), 0)
    tp_i = lax.broadcasted_iota(jnp.int32, (NWORK, NWORK), 1)
    shift_prev = ((t2_i % 2 == 1) & (tp_i == t2_i - 1)).astype(f32)
    prev_total = jnp.dot(shift_prev, nzero_h, precision=hp)
    prefix = prefix + prev_total
    mz_h = jnp.dot(expand, mz, precision=hp)
    selc = jnp.clip(mz_h - prefix, 0.0, zh) / jnp.maximum(zh, 1.0)
    s_z_h = jnp.sum(selc * zsh, axis=1, keepdims=True)
    s_z = jnp.dot(mpair, s_z_h, precision=hp)
    s_neg0 = s_gt0 + s_small + s_z

    s_neg = jnp.where(beta == 0.0, s_neg0, s_neg_hi)

    pos_cnt = jnp.maximum(jnp.sum(npos), 1.0)
    neg_cnt = jnp.maximum(jnp.sum(k), 1.0)
    loss = 2.0 * (jnp.sum(zsum) / pos_cnt) + jnp.sum(s_neg) / neg_cnt
    out_ref[0, 0] = loss


def _finish(cnt, fsum, npos):
    return pl.pallas_call(
        _finish_kernel,
        out_shape=jax.ShapeDtypeStruct((1, 1), jnp.float32),
        out_specs=pl.BlockSpec(memory_space=pltpu.SMEM),
    )(cnt.reshape(NWORK * NLANE, STRIDE), fsum.reshape(NWORK * NLANE, STRIDE),
      npos)


def kernel(y_pred, y_true):
    yp = y_pred.reshape(-1)
    yt = y_true.reshape(-1)
    cnt, fsum, npos = _sc_pass(yp, yt)
    out = _finish(cnt, fsum, npos)
    return jnp.reshape(out, ())
